# baseline (speedup 1.0000x reference)
import numpy as np

import jax
import jax.numpy as jnp
from jax.experimental import pallas as pl
from jax.experimental.pallas import tpu as pltpu

D_MODEL = 1024
MAX_LEN = 2048
SB = 128
NBUF = 6
CHUNK_SIZES = (128, 256, 640, 1024, 1024, 1024, 1024, 1024, 1024, 640, 256, 128)


def _pe_tables(max_len, d_model, s_blk):
    half = np.exp(np.arange(0, d_model, 2, dtype=np.float64) * (-np.log(10000.0) / d_model))
    d = np.repeat(half, 2)
    phi = np.zeros(d_model, dtype=np.float64)
    phi[1::2] = np.pi / 2.0
    g = np.arange(max_len // s_blk, dtype=np.float64)[:, None] * s_blk
    r = np.arange(s_blk, dtype=np.float64)[:, None]
    coarse_sin = np.sin(g * d).astype(np.float32)
    coarse_cos = np.cos(g * d).astype(np.float32)
    fine_sin = np.sin(r * d + phi).astype(np.float32)
    fine_cos = np.cos(r * d + phi).astype(np.float32)
    return coarse_sin, coarse_cos, fine_sin, fine_cos


def _body(x_hbm, f_ref, tab_ref, cs_ref, cc_ref, fs_ref, fc_ref, o_hbm,
          in_buf, out_buf, rsem, wsem):
    n_chunks = len(CHUNK_SIZES)
    bases = [0]
    for sz in CHUNK_SIZES[:-1]:
        bases.append(bases[-1] + sz)
    g_per_batch = MAX_LEN // SB

    def read_copy(c):
        slot = c % NBUF
        sz = CHUNK_SIZES[c]
        return pltpu.make_async_copy(
            x_hbm.at[pl.ds(bases[c], sz), :],
            in_buf.at[slot, pl.ds(0, sz), :], rsem.at[slot])

    def write_copy(c):
        slot = c % NBUF
        sz = CHUNK_SIZES[c]
        return pltpu.make_async_copy(
            out_buf.at[slot, pl.ds(0, sz), :],
            o_hbm.at[pl.ds(bases[c], sz), :], wsem.at[slot])

    for c in range(NBUF):
        read_copy(c).start()

    t0 = tab_ref[0, :]
    t1 = tab_ref[1, :]
    dv = 0.3 * (t1 - t0)
    base_add = 0.3 * t0

    for c in range(n_chunks):
        slot = c % NBUF
        read_copy(c).wait()
        if c >= NBUF:
            write_copy(c - NBUF).wait()
        for i in range(CHUNK_SIZES[c] // SB):
            k = bases[c] // SB + i
            g = k % g_per_batch
            pe = cs_ref[g, :] * fc_ref[...] + cc_ref[g, :] * fs_ref[...]
            f = jnp.clip(f_ref[k, 0, :], 0, 1).astype(jnp.float32)
            out_buf[slot, i * SB:(i + 1) * SB, :] = (
                in_buf[slot, i * SB:(i + 1) * SB, :]
                + (pe + base_add) + f[:, None] * dv)
        write_copy(c).start()
        if c + NBUF < n_chunks:
            read_copy(c + NBUF).start()

    for c in range(n_chunks - NBUF, n_chunks):
        write_copy(c).wait()


def kernel(x, crisis_flags, crisis_table):
    B, S, D = x.shape
    n_rows = B * S
    x2d = x.reshape(n_rows, D)
    flags = crisis_flags.astype(jnp.int32).reshape(n_rows // SB, 1, SB)
    cs, cc, fs, fc = _pe_tables(S, D, SB)
    out = pl.pallas_call(
        _body,
        in_specs=[
            pl.BlockSpec(memory_space=pltpu.HBM),
            pl.BlockSpec(memory_space=pltpu.VMEM),
            pl.BlockSpec(memory_space=pltpu.VMEM),
            pl.BlockSpec(memory_space=pltpu.VMEM),
            pl.BlockSpec(memory_space=pltpu.VMEM),
            pl.BlockSpec(memory_space=pltpu.VMEM),
            pl.BlockSpec(memory_space=pltpu.VMEM),
        ],
        out_specs=pl.BlockSpec(memory_space=pltpu.HBM),
        out_shape=jax.ShapeDtypeStruct((n_rows, D), x.dtype),
        scratch_shapes=[
            pltpu.VMEM((NBUF, max(CHUNK_SIZES), D), jnp.float32),
            pltpu.VMEM((NBUF, max(CHUNK_SIZES), D), jnp.float32),
            pltpu.SemaphoreType.DMA((NBUF,)),
            pltpu.SemaphoreType.DMA((NBUF,)),
        ],
    )(x2d, flags, crisis_table,
      jnp.asarray(cs), jnp.asarray(cc), jnp.asarray(fs), jnp.asarray(fc))
    return out.reshape(B, S, D)
